# Initial kernel scaffold; baseline (speedup 1.0000x reference)
import jax
import jax.numpy as jnp
from jax.experimental import pallas as pl


def kernel(x, edge_index, W1, b1, W2, b2):
    raise NotImplementedError("write your pallas kernel here")


# speedup vs baseline: 19.0188x; 19.0188x over previous
import jax
import jax.numpy as jnp
from jax import lax
from jax.experimental import pallas as pl
from jax.experimental.pallas import tpu as pltpu
from jax.experimental.pallas import tpu_sc as plsc

_NC = 2
_NS = 16
_NW = _NC * _NS
_B = 128

_MESH = plsc.VectorSubcoreMesh(
    core_axis_name="c", subcore_axis_name="s", num_cores=_NC, num_subcores=_NS
)


def _sc_deg(nb, npad):
    t_max = (nb + _NW - 1) // _NW

    def body(dst_hbm, z_hbm, out0_hbm, out1_hbm, idx_v, ones_v, acc, sem):
        c = lax.axis_index("c")
        s = lax.axis_index("s")
        wid = s * _NC + c
        for k in range(_B // 16):
            ones_v[pl.ds(16 * k, 16)] = jnp.full((16,), 1.0, jnp.float32)

        @pl.when(s == 0)
        def _init():
            pltpu.sync_copy(z_hbm, acc)

        plsc.subcore_barrier()

        def step(t, carry):
            j = wid + _NW * t

            @pl.when(j < nb)
            def _go():
                pltpu.sync_copy(dst_hbm.at[j], idx_v)
                pltpu.sync_copy(ones_v, acc.at[idx_v], add=True)

            return carry

        lax.fori_loop(0, t_max, step, 0)
        plsc.subcore_barrier()

        @pl.when((s == 0) & (c == 0))
        def _w0():
            pltpu.sync_copy(acc, out0_hbm)

        @pl.when((s == 0) & (c == 1))
        def _w1():
            pltpu.sync_copy(acc, out1_hbm)

    return pl.kernel(
        body,
        out_type=[
            jax.ShapeDtypeStruct((npad,), jnp.float32),
            jax.ShapeDtypeStruct((npad,), jnp.float32),
        ],
        mesh=_MESH,
        scratch_types=[
            pltpu.VMEM((_B,), jnp.int32),
            pltpu.VMEM((_B,), jnp.float32),
            pltpu.VMEM_SHARED((npad,), jnp.float32),
            pltpu.SemaphoreType.DMA,
        ],
    )


def _sc_agg(nb, npad, d):
    t_max = (nb + _NW - 1) // _NW
    wide = d > 1
    acc_shape = (npad, d) if wide else (npad,)
    row_shape = (_B, d) if wide else (_B,)

    def body(src_hbm, dst_hbm, g_hbm, z_hbm, out0_hbm, out1_hbm,
             idx_s, idx_d, rows, acc, sem):
        c = lax.axis_index("c")
        s = lax.axis_index("s")
        wid = s * _NC + c

        @pl.when(s == 0)
        def _init():
            pltpu.sync_copy(z_hbm, acc)

        plsc.subcore_barrier()

        def step(t, carry):
            j = wid + _NW * t

            @pl.when(j < nb)
            def _go():
                pltpu.sync_copy(src_hbm.at[j], idx_s)
                pltpu.sync_copy(dst_hbm.at[j], idx_d)
                pltpu.async_copy(g_hbm.at[idx_s], rows, sem).wait()
                pltpu.sync_copy(rows, acc.at[idx_d], add=True)

            return carry

        lax.fori_loop(0, t_max, step, 0)
        plsc.subcore_barrier()

        @pl.when((s == 0) & (c == 0))
        def _w0():
            pltpu.sync_copy(acc, out0_hbm)

        @pl.when((s == 0) & (c == 1))
        def _w1():
            pltpu.sync_copy(acc, out1_hbm)

    out_shape = (npad, d) if wide else (npad,)
    return pl.kernel(
        body,
        out_type=[
            jax.ShapeDtypeStruct(out_shape, jnp.float32),
            jax.ShapeDtypeStruct(out_shape, jnp.float32),
        ],
        mesh=_MESH,
        scratch_types=[
            pltpu.VMEM((_B,), jnp.int32),
            pltpu.VMEM((_B,), jnp.int32),
            pltpu.VMEM(row_shape, jnp.float32),
            pltpu.VMEM_SHARED(acc_shape, jnp.float32),
            pltpu.SemaphoreType.DMA,
        ],
    )


def _tc1_body(x_ref, w1_ref, d0_ref, d1_ref, g1_ref, dinv_ref):
    deg = d0_ref[...] + d1_ref[...] + 1.0
    dinv = lax.rsqrt(deg)
    dinv_ref[...] = dinv
    h = jnp.dot(x_ref[...], w1_ref[...], preferred_element_type=jnp.float32)
    g1_ref[...] = dinv * h


def _tc2_body(a0_ref, a1_ref, g1_ref, dinv_ref, b1_ref, w2_ref, g2_ref):
    dinv = dinv_ref[...]
    out1 = dinv * (a0_ref[...] + a1_ref[...] + g1_ref[...]) + b1_ref[...]
    out1 = jnp.maximum(out1, 0.0)
    h2 = jnp.sum(out1 * w2_ref[...], axis=1, keepdims=True)
    g2_ref[...] = dinv * h2


def _tc3_body(a0_ref, a1_ref, g2_ref, dinv_ref, b2_ref, out_ref):
    out_ref[...] = (
        dinv_ref[...] * (a0_ref[...] + a1_ref[...] + g2_ref[...]) + b2_ref[...]
    )


def kernel(x, edge_index, W1, b1, W2, b2):
    n, d_in = x.shape
    d_h = W1.shape[1]
    dp = 128
    e = edge_index.shape[1]
    nb = e // _B
    npad = -(-n // 128) * 128
    rb = 1000
    grid = (n // rb,)

    src2d = edge_index[0].reshape(nb, _B)
    dst2d = edge_index[1].reshape(nb, _B)
    z1 = jnp.zeros((npad,), jnp.float32)
    zp = jnp.zeros((npad, dp), jnp.float32)
    W1p = jnp.zeros((d_in, dp), W1.dtype).at[:, :d_h].set(W1)

    degp = _sc_deg(nb, npad)(dst2d, z1)
    d0 = degp[0][:n, None]
    d1 = degp[1][:n, None]

    g1, dinv = pl.pallas_call(
        _tc1_body,
        grid=grid,
        in_specs=[
            pl.BlockSpec((rb, d_in), lambda i: (i, 0)),
            pl.BlockSpec((d_in, dp), lambda i: (0, 0)),
            pl.BlockSpec((rb, 1), lambda i: (i, 0)),
            pl.BlockSpec((rb, 1), lambda i: (i, 0)),
        ],
        out_specs=[
            pl.BlockSpec((rb, dp), lambda i: (i, 0)),
            pl.BlockSpec((rb, 1), lambda i: (i, 0)),
        ],
        out_shape=[
            jax.ShapeDtypeStruct((n, dp), jnp.float32),
            jax.ShapeDtypeStruct((n, 1), jnp.float32),
        ],
    )(x, W1p, d0, d1)

    a0, a1 = _sc_agg(nb, npad, dp)(src2d, dst2d, g1, zp)

    b1p = jnp.zeros((1, dp), jnp.float32).at[0, :d_h].set(b1)
    W2p = jnp.zeros((1, dp), jnp.float32).at[0, :d_h].set(W2[:, 0])
    g2 = pl.pallas_call(
        _tc2_body,
        grid=grid,
        in_specs=[
            pl.BlockSpec((rb, dp), lambda i: (i, 0)),
            pl.BlockSpec((rb, dp), lambda i: (i, 0)),
            pl.BlockSpec((rb, dp), lambda i: (i, 0)),
            pl.BlockSpec((rb, 1), lambda i: (i, 0)),
            pl.BlockSpec((1, dp), lambda i: (0, 0)),
            pl.BlockSpec((1, dp), lambda i: (0, 0)),
        ],
        out_specs=pl.BlockSpec((rb, 1), lambda i: (i, 0)),
        out_shape=jax.ShapeDtypeStruct((n, 1), jnp.float32),
    )(a0, a1, g1, dinv, b1p, W2p)

    s0, s1 = _sc_agg(nb, npad, 1)(src2d, dst2d, g2.reshape(n), z1)

    out = pl.pallas_call(
        _tc3_body,
        grid=grid,
        in_specs=[
            pl.BlockSpec((rb, 1), lambda i: (i, 0)),
            pl.BlockSpec((rb, 1), lambda i: (i, 0)),
            pl.BlockSpec((rb, 1), lambda i: (i, 0)),
            pl.BlockSpec((rb, 1), lambda i: (i, 0)),
            pl.BlockSpec((1, 1), lambda i: (0, 0)),
        ],
        out_specs=pl.BlockSpec((rb, 1), lambda i: (i, 0)),
        out_shape=jax.ShapeDtypeStruct((n, 1), jnp.float32),
    )(s0[:n, None], s1[:n, None], g2, dinv, b2.reshape(1, 1))

    return out


# speedup vs baseline: 30.6630x; 1.6122x over previous
import jax
import jax.numpy as jnp
from jax import lax
from jax.experimental import pallas as pl
from jax.experimental.pallas import tpu as pltpu
from jax.experimental.pallas import tpu_sc as plsc

_NC = 2
_NS = 16
_NW = _NC * _NS
_B = 128

_MESH = plsc.VectorSubcoreMesh(
    core_axis_name="c", subcore_axis_name="s", num_cores=_NC, num_subcores=_NS
)


def _sc_deg(nb, npad):
    t_max = (nb + _NW - 1) // _NW

    def body(dst_hbm, z_hbm, out0_hbm, out1_hbm, idx_v, ones_v, acc, sem):
        c = lax.axis_index("c")
        s = lax.axis_index("s")
        wid = s * _NC + c
        for k in range(_B // 16):
            ones_v[pl.ds(16 * k, 16)] = jnp.full((16,), 1.0, jnp.float32)

        @pl.when(s == 0)
        def _init():
            pltpu.sync_copy(z_hbm, acc)

        plsc.subcore_barrier()

        def step(t, carry):
            j = wid + _NW * t

            @pl.when(j < nb)
            def _go():
                pltpu.sync_copy(dst_hbm.at[j], idx_v)
                pltpu.sync_copy(ones_v, acc.at[idx_v], add=True)

            return carry

        lax.fori_loop(0, t_max, step, 0)
        plsc.subcore_barrier()

        @pl.when((s == 0) & (c == 0))
        def _w0():
            pltpu.sync_copy(acc, out0_hbm)

        @pl.when((s == 0) & (c == 1))
        def _w1():
            pltpu.sync_copy(acc, out1_hbm)

    return pl.kernel(
        body,
        out_type=[
            jax.ShapeDtypeStruct((npad,), jnp.float32),
            jax.ShapeDtypeStruct((npad,), jnp.float32),
        ],
        mesh=_MESH,
        scratch_types=[
            pltpu.VMEM((_B,), jnp.int32),
            pltpu.VMEM((_B,), jnp.float32),
            pltpu.VMEM_SHARED((npad,), jnp.float32),
            pltpu.SemaphoreType.DMA,
        ],
    )


def _sc_agg(nb, npad, d):
    t_max = (nb + _NW - 1) // _NW
    nbuf = 2
    t_out = (t_max + nbuf - 1) // nbuf
    wide = d > 1
    acc_shape = (npad, d) if wide else (npad,)
    rows_shape = (nbuf, _B, d) if wide else (nbuf, _B)

    def body(ep_hbm, g_hbm, z_hbm, out0_hbm, out1_hbm,
             idx2, rows, acc, sem0, sem1):
        c = lax.axis_index("c")
        s = lax.axis_index("s")
        wid = s * _NC + c
        sems = [sem0, sem1]

        @pl.when(s == 0)
        def _init():
            pltpu.sync_copy(z_hbm, acc)

        plsc.subcore_barrier()

        for b in range(nbuf):
            jp = wid + _NW * b

            @pl.when(jp < nb)
            def _prime(b=b, jp=jp):
                pltpu.sync_copy(ep_hbm.at[jp], idx2.at[b])
                pltpu.async_copy(g_hbm.at[idx2.at[b, 0]], rows.at[b], sems[b])

        def step(ti, carry):
            tt = ti * nbuf
            for b in range(nbuf):
                j = wid + _NW * (tt + b)

                @pl.when(j < nb)
                def _drain(b=b):
                    pltpu.make_async_copy(
                        g_hbm.at[idx2.at[b, 0]], rows.at[b], sems[b]
                    ).wait()
                    pltpu.sync_copy(rows.at[b], acc.at[idx2.at[b, 1]], add=True)

                jn = j + _NW * nbuf

                @pl.when(jn < nb)
                def _next(b=b, jn=jn):
                    pltpu.sync_copy(ep_hbm.at[jn], idx2.at[b])
                    pltpu.async_copy(g_hbm.at[idx2.at[b, 0]], rows.at[b], sems[b])

            return carry

        lax.fori_loop(0, t_out, step, 0)
        plsc.subcore_barrier()

        @pl.when((s == 0) & (c == 0))
        def _w0():
            pltpu.sync_copy(acc, out0_hbm)

        @pl.when((s == 0) & (c == 1))
        def _w1():
            pltpu.sync_copy(acc, out1_hbm)

    out_shape = (npad, d) if wide else (npad,)
    return pl.kernel(
        body,
        out_type=[
            jax.ShapeDtypeStruct(out_shape, jnp.float32),
            jax.ShapeDtypeStruct(out_shape, jnp.float32),
        ],
        mesh=_MESH,
        scratch_types=[
            pltpu.VMEM((nbuf, 2, _B), jnp.int32),
            pltpu.VMEM(rows_shape, jnp.float32),
            pltpu.VMEM_SHARED(acc_shape, jnp.float32),
            pltpu.SemaphoreType.DMA,
            pltpu.SemaphoreType.DMA,
        ],
    )


def _tc1_body(x_ref, w1_ref, d0_ref, d1_ref, g1_ref, dinv_ref):
    deg = d0_ref[...] + d1_ref[...] + 1.0
    dinv = lax.rsqrt(deg)
    dinv_ref[...] = dinv
    h = jnp.dot(x_ref[...], w1_ref[...], preferred_element_type=jnp.float32)
    g1_ref[...] = dinv * h


def _tc2_body(a0_ref, a1_ref, g1_ref, dinv_ref, b1_ref, w2_ref, g2_ref):
    dinv = dinv_ref[...]
    out1 = dinv * (a0_ref[...] + a1_ref[...] + g1_ref[...]) + b1_ref[...]
    out1 = jnp.maximum(out1, 0.0)
    h2 = jnp.sum(out1 * w2_ref[...], axis=1, keepdims=True)
    g2_ref[...] = dinv * h2


def _tc3_body(a0_ref, a1_ref, g2_ref, dinv_ref, b2_ref, out_ref):
    out_ref[...] = (
        dinv_ref[...] * (a0_ref[...] + a1_ref[...] + g2_ref[...]) + b2_ref[...]
    )


def kernel(x, edge_index, W1, b1, W2, b2):
    n, d_in = x.shape
    d_h = W1.shape[1]
    dp = 128
    e = edge_index.shape[1]
    nb = e // _B
    npad = -(-n // 128) * 128
    rb = 1000
    grid = (n // rb,)

    src2d = edge_index[0].reshape(nb, _B)
    dst2d = edge_index[1].reshape(nb, _B)
    ep = jnp.stack([src2d, dst2d], axis=1)
    z1 = jnp.zeros((npad,), jnp.float32)
    zp = jnp.zeros((npad, dp), jnp.float32)
    W1p = jnp.zeros((d_in, dp), W1.dtype).at[:, :d_h].set(W1)

    degp = _sc_deg(nb, npad)(dst2d, z1)
    d0 = degp[0][:n, None]
    d1 = degp[1][:n, None]

    g1, dinv = pl.pallas_call(
        _tc1_body,
        grid=grid,
        in_specs=[
            pl.BlockSpec((rb, d_in), lambda i: (i, 0)),
            pl.BlockSpec((d_in, dp), lambda i: (0, 0)),
            pl.BlockSpec((rb, 1), lambda i: (i, 0)),
            pl.BlockSpec((rb, 1), lambda i: (i, 0)),
        ],
        out_specs=[
            pl.BlockSpec((rb, dp), lambda i: (i, 0)),
            pl.BlockSpec((rb, 1), lambda i: (i, 0)),
        ],
        out_shape=[
            jax.ShapeDtypeStruct((n, dp), jnp.float32),
            jax.ShapeDtypeStruct((n, 1), jnp.float32),
        ],
    )(x, W1p, d0, d1)

    a0, a1 = _sc_agg(nb, npad, dp)(ep, g1, zp)

    b1p = jnp.zeros((1, dp), jnp.float32).at[0, :d_h].set(b1)
    W2p = jnp.zeros((1, dp), jnp.float32).at[0, :d_h].set(W2[:, 0])
    g2 = pl.pallas_call(
        _tc2_body,
        grid=grid,
        in_specs=[
            pl.BlockSpec((rb, dp), lambda i: (i, 0)),
            pl.BlockSpec((rb, dp), lambda i: (i, 0)),
            pl.BlockSpec((rb, dp), lambda i: (i, 0)),
            pl.BlockSpec((rb, 1), lambda i: (i, 0)),
            pl.BlockSpec((1, dp), lambda i: (0, 0)),
            pl.BlockSpec((1, dp), lambda i: (0, 0)),
        ],
        out_specs=pl.BlockSpec((rb, 1), lambda i: (i, 0)),
        out_shape=jax.ShapeDtypeStruct((n, 1), jnp.float32),
    )(a0, a1, g1, dinv, b1p, W2p)

    s0, s1 = _sc_agg(nb, npad, 1)(ep, g2.reshape(n), z1)

    out = pl.pallas_call(
        _tc3_body,
        grid=grid,
        in_specs=[
            pl.BlockSpec((rb, 1), lambda i: (i, 0)),
            pl.BlockSpec((rb, 1), lambda i: (i, 0)),
            pl.BlockSpec((rb, 1), lambda i: (i, 0)),
            pl.BlockSpec((rb, 1), lambda i: (i, 0)),
            pl.BlockSpec((1, 1), lambda i: (0, 0)),
        ],
        out_specs=pl.BlockSpec((rb, 1), lambda i: (i, 0)),
        out_shape=jax.ShapeDtypeStruct((n, 1), jnp.float32),
    )(s0[:n, None], s1[:n, None], g2, dinv, b2.reshape(1, 1))

    return out


# speedup vs baseline: 33.0673x; 1.0784x over previous
import jax
import jax.numpy as jnp
from jax import lax
from jax.experimental import pallas as pl
from jax.experimental.pallas import tpu as pltpu
from jax.experimental.pallas import tpu_sc as plsc

_NC = 2
_NS = 16
_NW = _NC * _NS
_B = 128

_MESH = plsc.VectorSubcoreMesh(
    core_axis_name="c", subcore_axis_name="s", num_cores=_NC, num_subcores=_NS
)


def _sc_deg(nb, npad):
    t_max = (nb + _NW - 1) // _NW
    nbuf = 2
    t_out = (t_max + nbuf - 1) // nbuf

    def body(dst_hbm, z_hbm, out0_hbm, out1_hbm, idxb, ones_v, acc,
             sem0, sem1):
        c = lax.axis_index("c")
        s = lax.axis_index("s")
        wid = s * _NC + c
        sems = [sem0, sem1]
        for k in range(_B // 16):
            ones_v[pl.ds(16 * k, 16)] = jnp.full((16,), 1.0, jnp.float32)

        @pl.when(s == 0)
        def _init():
            pltpu.sync_copy(z_hbm, acc)

        plsc.subcore_barrier()

        for b in range(nbuf):
            jp = wid + _NW * b

            @pl.when(jp < nb)
            def _prime(b=b, jp=jp):
                pltpu.async_copy(dst_hbm.at[jp], idxb.at[b, 0], sems[b])

        def step(ti, carry):
            tt = ti * nbuf
            for b in range(nbuf):
                j = wid + _NW * (tt + b)

                @pl.when(j < nb)
                def _go(b=b):
                    pltpu.make_async_copy(
                        dst_hbm.at[j], idxb.at[b, 0], sems[b]
                    ).wait()
                    pltpu.sync_copy(ones_v, acc.at[idxb.at[b, 0]], add=True)

                jn = j + _NW * nbuf

                @pl.when(jn < nb)
                def _next(b=b, jn=jn):
                    pltpu.async_copy(dst_hbm.at[jn], idxb.at[b, 0], sems[b])

            return carry

        lax.fori_loop(0, t_out, step, 0)
        plsc.subcore_barrier()

        @pl.when((s == 0) & (c == 0))
        def _w0():
            pltpu.sync_copy(acc, out0_hbm)

        @pl.when((s == 0) & (c == 1))
        def _w1():
            pltpu.sync_copy(acc, out1_hbm)

    return pl.kernel(
        body,
        out_type=[
            jax.ShapeDtypeStruct((npad,), jnp.float32),
            jax.ShapeDtypeStruct((npad,), jnp.float32),
        ],
        mesh=_MESH,
        scratch_types=[
            pltpu.VMEM((nbuf, 1, _B), jnp.int32),
            pltpu.VMEM((_B,), jnp.float32),
            pltpu.VMEM_SHARED((npad,), jnp.float32),
            pltpu.SemaphoreType.DMA,
            pltpu.SemaphoreType.DMA,
        ],
    )


def _sc_agg(nb, npad, d):
    t_max = (nb + _NW - 1) // _NW
    nbuf = 2
    t_out = (t_max + nbuf - 1) // nbuf
    wide = d > 1
    acc_shape = (npad, d) if wide else (npad,)
    rows_shape = (nbuf, _B, d) if wide else (nbuf, _B)

    def body(ep_hbm, g_hbm, z_hbm, out0_hbm, out1_hbm,
             idx2, rows, acc, sem0, sem1):
        c = lax.axis_index("c")
        s = lax.axis_index("s")
        wid = s * _NC + c
        sems = [sem0, sem1]

        @pl.when(s == 0)
        def _init():
            pltpu.sync_copy(z_hbm, acc)

        plsc.subcore_barrier()

        for b in range(nbuf):
            jp = wid + _NW * b

            @pl.when(jp < nb)
            def _prime(b=b, jp=jp):
                pltpu.sync_copy(ep_hbm.at[jp], idx2.at[b])
                pltpu.async_copy(g_hbm.at[idx2.at[b, 0]], rows.at[b], sems[b])

        def step(ti, carry):
            tt = ti * nbuf
            for b in range(nbuf):
                j = wid + _NW * (tt + b)

                @pl.when(j < nb)
                def _drain(b=b):
                    pltpu.make_async_copy(
                        g_hbm.at[idx2.at[b, 0]], rows.at[b], sems[b]
                    ).wait()
                    pltpu.sync_copy(rows.at[b], acc.at[idx2.at[b, 1]], add=True)

                jn = j + _NW * nbuf

                @pl.when(jn < nb)
                def _next(b=b, jn=jn):
                    pltpu.sync_copy(ep_hbm.at[jn], idx2.at[b])
                    pltpu.async_copy(g_hbm.at[idx2.at[b, 0]], rows.at[b], sems[b])

            return carry

        lax.fori_loop(0, t_out, step, 0)
        plsc.subcore_barrier()

        @pl.when((s == 0) & (c == 0))
        def _w0():
            pltpu.sync_copy(acc, out0_hbm)

        @pl.when((s == 0) & (c == 1))
        def _w1():
            pltpu.sync_copy(acc, out1_hbm)

    out_shape = (npad, d) if wide else (npad,)
    return pl.kernel(
        body,
        out_type=[
            jax.ShapeDtypeStruct(out_shape, jnp.float32),
            jax.ShapeDtypeStruct(out_shape, jnp.float32),
        ],
        mesh=_MESH,
        scratch_types=[
            pltpu.VMEM((nbuf, 2, _B), jnp.int32),
            pltpu.VMEM(rows_shape, jnp.float32),
            pltpu.VMEM_SHARED(acc_shape, jnp.float32),
            pltpu.SemaphoreType.DMA,
            pltpu.SemaphoreType.DMA,
        ],
    )


def _tc_mm_body(x_ref, w1_ref, h_ref):
    h_ref[...] = jnp.dot(
        x_ref[...], w1_ref[...], preferred_element_type=jnp.float32
    )


def _tc_scale_body(h_ref, d0_ref, d1_ref, g1_ref, dinv_ref):
    deg = d0_ref[...] + d1_ref[...] + 1.0
    dinv = lax.rsqrt(deg)
    dinv_ref[...] = dinv
    g1_ref[...] = dinv * h_ref[...]


def _tc2_body(a0_ref, a1_ref, g1_ref, dinv_ref, b1_ref, w2_ref, g2_ref):
    dinv = dinv_ref[...]
    out1 = dinv * (a0_ref[...] + a1_ref[...] + g1_ref[...]) + b1_ref[...]
    out1 = jnp.maximum(out1, 0.0)
    h2 = jnp.sum(out1 * w2_ref[...], axis=1, keepdims=True)
    g2_ref[...] = dinv * h2


def _tc3_body(a0_ref, a1_ref, g2_ref, dinv_ref, b2_ref, out_ref):
    out_ref[...] = (
        dinv_ref[...] * (a0_ref[...] + a1_ref[...] + g2_ref[...]) + b2_ref[...]
    )


def kernel(x, edge_index, W1, b1, W2, b2):
    n, d_in = x.shape
    d_h = W1.shape[1]
    dp = 128
    e = edge_index.shape[1]
    nb = e // _B
    npad = -(-n // 128) * 128
    rb = 1000
    grid = (n // rb,)

    src2d = edge_index[0].reshape(nb, _B)
    dst2d = edge_index[1].reshape(nb, _B)
    ep = jnp.stack([src2d, dst2d], axis=1)
    z1 = jnp.zeros((npad,), jnp.float32)
    zp = jnp.zeros((npad, dp), jnp.float32)
    W1p = jnp.zeros((d_in, dp), W1.dtype).at[:, :d_h].set(W1)

    degp = _sc_deg(nb, npad)(dst2d, z1)
    h = pl.pallas_call(
        _tc_mm_body,
        grid=grid,
        in_specs=[
            pl.BlockSpec((rb, d_in), lambda i: (i, 0)),
            pl.BlockSpec((d_in, dp), lambda i: (0, 0)),
        ],
        out_specs=pl.BlockSpec((rb, dp), lambda i: (i, 0)),
        out_shape=jax.ShapeDtypeStruct((n, dp), jnp.float32),
    )(x, W1p)
    d0 = degp[0][:n, None]
    d1 = degp[1][:n, None]

    g1, dinv = pl.pallas_call(
        _tc_scale_body,
        grid=grid,
        in_specs=[
            pl.BlockSpec((rb, dp), lambda i: (i, 0)),
            pl.BlockSpec((rb, 1), lambda i: (i, 0)),
            pl.BlockSpec((rb, 1), lambda i: (i, 0)),
        ],
        out_specs=[
            pl.BlockSpec((rb, dp), lambda i: (i, 0)),
            pl.BlockSpec((rb, 1), lambda i: (i, 0)),
        ],
        out_shape=[
            jax.ShapeDtypeStruct((n, dp), jnp.float32),
            jax.ShapeDtypeStruct((n, 1), jnp.float32),
        ],
    )(h, d0, d1)

    a0, a1 = _sc_agg(nb, npad, dp)(ep, g1, zp)

    b1p = jnp.zeros((1, dp), jnp.float32).at[0, :d_h].set(b1)
    W2p = jnp.zeros((1, dp), jnp.float32).at[0, :d_h].set(W2[:, 0])
    g2 = pl.pallas_call(
        _tc2_body,
        grid=grid,
        in_specs=[
            pl.BlockSpec((rb, dp), lambda i: (i, 0)),
            pl.BlockSpec((rb, dp), lambda i: (i, 0)),
            pl.BlockSpec((rb, dp), lambda i: (i, 0)),
            pl.BlockSpec((rb, 1), lambda i: (i, 0)),
            pl.BlockSpec((1, dp), lambda i: (0, 0)),
            pl.BlockSpec((1, dp), lambda i: (0, 0)),
        ],
        out_specs=pl.BlockSpec((rb, 1), lambda i: (i, 0)),
        out_shape=jax.ShapeDtypeStruct((n, 1), jnp.float32),
    )(a0, a1, g1, dinv, b1p, W2p)

    s0, s1 = _sc_agg(nb, npad, 1)(ep, g2.reshape(n), z1)

    out = pl.pallas_call(
        _tc3_body,
        grid=grid,
        in_specs=[
            pl.BlockSpec((rb, 1), lambda i: (i, 0)),
            pl.BlockSpec((rb, 1), lambda i: (i, 0)),
            pl.BlockSpec((rb, 1), lambda i: (i, 0)),
            pl.BlockSpec((rb, 1), lambda i: (i, 0)),
            pl.BlockSpec((1, 1), lambda i: (0, 0)),
        ],
        out_specs=pl.BlockSpec((rb, 1), lambda i: (i, 0)),
        out_shape=jax.ShapeDtypeStruct((n, 1), jnp.float32),
    )(s0[:n, None], s1[:n, None], g2, dinv, b2.reshape(1, 1))

    return out


# speedup vs baseline: 33.9230x; 1.0259x over previous
import jax
import jax.numpy as jnp
from jax import lax
from jax.experimental import pallas as pl
from jax.experimental.pallas import tpu as pltpu
from jax.experimental.pallas import tpu_sc as plsc

_NC = 2
_NS = 16
_NW = _NC * _NS
_B = 128

_MESH = plsc.VectorSubcoreMesh(
    core_axis_name="c", subcore_axis_name="s", num_cores=_NC, num_subcores=_NS
)


def _sc_deg(nb, npad):
    t_max = (nb + _NW - 1) // _NW
    nbuf = 2
    t_out = (t_max + nbuf - 1) // nbuf

    def body(dst_hbm, z_hbm, out0_hbm, out1_hbm, idxb, ones_v, acc,
             sem0, sem1):
        c = lax.axis_index("c")
        s = lax.axis_index("s")
        wid = s * _NC + c
        sems = [sem0, sem1]
        for k in range(_B // 16):
            ones_v[pl.ds(16 * k, 16)] = jnp.full((16,), 1.0, jnp.float32)

        @pl.when(s == 0)
        def _init():
            pltpu.sync_copy(z_hbm, acc)

        plsc.subcore_barrier()

        for b in range(nbuf):
            jp = wid + _NW * b

            @pl.when(jp < nb)
            def _prime(b=b, jp=jp):
                pltpu.async_copy(dst_hbm.at[jp], idxb.at[b, 0], sems[b])

        def step(ti, carry):
            tt = ti * nbuf
            for b in range(nbuf):
                j = wid + _NW * (tt + b)

                @pl.when(j < nb)
                def _go(b=b):
                    pltpu.make_async_copy(
                        dst_hbm.at[j], idxb.at[b, 0], sems[b]
                    ).wait()
                    pltpu.sync_copy(ones_v, acc.at[idxb.at[b, 0]], add=True)

                jn = j + _NW * nbuf

                @pl.when(jn < nb)
                def _next(b=b, jn=jn):
                    pltpu.async_copy(dst_hbm.at[jn], idxb.at[b, 0], sems[b])

            return carry

        lax.fori_loop(0, t_out, step, 0)
        plsc.subcore_barrier()

        @pl.when((s == 0) & (c == 0))
        def _w0():
            pltpu.sync_copy(acc, out0_hbm)

        @pl.when((s == 0) & (c == 1))
        def _w1():
            pltpu.sync_copy(acc, out1_hbm)

    return pl.kernel(
        body,
        out_type=[
            jax.ShapeDtypeStruct((npad,), jnp.float32),
            jax.ShapeDtypeStruct((npad,), jnp.float32),
        ],
        mesh=_MESH,
        scratch_types=[
            pltpu.VMEM((nbuf, 1, _B), jnp.int32),
            pltpu.VMEM((_B,), jnp.float32),
            pltpu.VMEM_SHARED((npad,), jnp.float32),
            pltpu.SemaphoreType.DMA,
            pltpu.SemaphoreType.DMA,
        ],
    )


def _sc_agg(nb, npad, d):
    t_max = (nb + _NW - 1) // _NW
    nbuf = 2 if d > 1 else 4
    t_out = (t_max + nbuf - 1) // nbuf
    wide = d > 1
    acc_shape = (npad, d) if wide else (npad,)
    rows_shape = (nbuf, _B, d) if wide else (nbuf, _B)

    def body(ep_hbm, g_hbm, z_hbm, out0_hbm, out1_hbm,
             idx2, rows, acc, *sems):
        c = lax.axis_index("c")
        s = lax.axis_index("s")
        wid = s * _NC + c

        @pl.when(s == 0)
        def _init():
            pltpu.sync_copy(z_hbm, acc)

        plsc.subcore_barrier()

        for b in range(nbuf):
            jp = wid + _NW * b

            @pl.when(jp < nb)
            def _prime(b=b, jp=jp):
                pltpu.sync_copy(ep_hbm.at[jp], idx2.at[b])
                pltpu.async_copy(g_hbm.at[idx2.at[b, 0]], rows.at[b], sems[b])

        def step(ti, carry):
            tt = ti * nbuf
            for b in range(nbuf):
                j = wid + _NW * (tt + b)

                @pl.when(j < nb)
                def _drain(b=b):
                    pltpu.make_async_copy(
                        g_hbm.at[idx2.at[b, 0]], rows.at[b], sems[b]
                    ).wait()
                    pltpu.sync_copy(rows.at[b], acc.at[idx2.at[b, 1]], add=True)

                jn = j + _NW * nbuf

                @pl.when(jn < nb)
                def _next(b=b, jn=jn):
                    pltpu.sync_copy(ep_hbm.at[jn], idx2.at[b])
                    pltpu.async_copy(g_hbm.at[idx2.at[b, 0]], rows.at[b], sems[b])

            return carry

        lax.fori_loop(0, t_out, step, 0)
        plsc.subcore_barrier()

        @pl.when((s == 0) & (c == 0))
        def _w0():
            pltpu.sync_copy(acc, out0_hbm)

        @pl.when((s == 0) & (c == 1))
        def _w1():
            pltpu.sync_copy(acc, out1_hbm)

    out_shape = (npad, d) if wide else (npad,)
    return pl.kernel(
        body,
        out_type=[
            jax.ShapeDtypeStruct(out_shape, jnp.float32),
            jax.ShapeDtypeStruct(out_shape, jnp.float32),
        ],
        mesh=_MESH,
        scratch_types=[
            pltpu.VMEM((nbuf, 2, _B), jnp.int32),
            pltpu.VMEM(rows_shape, jnp.float32),
            pltpu.VMEM_SHARED(acc_shape, jnp.float32),
        ] + [pltpu.SemaphoreType.DMA] * nbuf,
    )


def _tc_mm_body(x_ref, w1_ref, h_ref):
    h_ref[...] = jnp.dot(
        x_ref[...], w1_ref[...], preferred_element_type=jnp.float32
    )


def _tc_scale_body(h_ref, d0_ref, d1_ref, g1_ref, dinv_ref):
    deg = d0_ref[...] + d1_ref[...] + 1.0
    dinv = lax.rsqrt(deg)
    dinv_ref[...] = dinv
    g1_ref[...] = dinv * h_ref[...]


def _tc2_body(a0_ref, a1_ref, g1_ref, dinv_ref, b1_ref, w2_ref, g2_ref):
    dinv = dinv_ref[...]
    out1 = dinv * (a0_ref[...] + a1_ref[...] + g1_ref[...]) + b1_ref[...]
    out1 = jnp.maximum(out1, 0.0)
    h2 = jnp.sum(out1 * w2_ref[...], axis=1, keepdims=True)
    g2_ref[...] = dinv * h2


def _tc3_body(a0_ref, a1_ref, g2_ref, dinv_ref, b2_ref, out_ref):
    out_ref[...] = (
        dinv_ref[...] * (a0_ref[...] + a1_ref[...] + g2_ref[...]) + b2_ref[...]
    )


def kernel(x, edge_index, W1, b1, W2, b2):
    n, d_in = x.shape
    d_h = W1.shape[1]
    dp = 128
    e = edge_index.shape[1]
    nb = e // _B
    npad = -(-n // 128) * 128
    rb = 1000
    grid = (n // rb,)

    src2d = edge_index[0].reshape(nb, _B)
    dst2d = edge_index[1].reshape(nb, _B)
    ep = jnp.stack([src2d, dst2d], axis=1)
    z1 = jnp.zeros((npad,), jnp.float32)
    zp = jnp.zeros((npad, dp), jnp.float32)
    W1p = jnp.zeros((d_in, dp), W1.dtype).at[:, :d_h].set(W1)

    degp = _sc_deg(nb, npad)(dst2d, z1)
    h = pl.pallas_call(
        _tc_mm_body,
        grid=grid,
        in_specs=[
            pl.BlockSpec((rb, d_in), lambda i: (i, 0)),
            pl.BlockSpec((d_in, dp), lambda i: (0, 0)),
        ],
        out_specs=pl.BlockSpec((rb, dp), lambda i: (i, 0)),
        out_shape=jax.ShapeDtypeStruct((n, dp), jnp.float32),
    )(x, W1p)
    d0 = degp[0][:n, None]
    d1 = degp[1][:n, None]

    g1, dinv = pl.pallas_call(
        _tc_scale_body,
        grid=grid,
        in_specs=[
            pl.BlockSpec((rb, dp), lambda i: (i, 0)),
            pl.BlockSpec((rb, 1), lambda i: (i, 0)),
            pl.BlockSpec((rb, 1), lambda i: (i, 0)),
        ],
        out_specs=[
            pl.BlockSpec((rb, dp), lambda i: (i, 0)),
            pl.BlockSpec((rb, 1), lambda i: (i, 0)),
        ],
        out_shape=[
            jax.ShapeDtypeStruct((n, dp), jnp.float32),
            jax.ShapeDtypeStruct((n, 1), jnp.float32),
        ],
    )(h, d0, d1)

    a0, a1 = _sc_agg(nb, npad, dp)(ep, g1, zp)

    b1p = jnp.zeros((1, dp), jnp.float32).at[0, :d_h].set(b1)
    W2p = jnp.zeros((1, dp), jnp.float32).at[0, :d_h].set(W2[:, 0])
    g2 = pl.pallas_call(
        _tc2_body,
        grid=grid,
        in_specs=[
            pl.BlockSpec((rb, dp), lambda i: (i, 0)),
            pl.BlockSpec((rb, dp), lambda i: (i, 0)),
            pl.BlockSpec((rb, dp), lambda i: (i, 0)),
            pl.BlockSpec((rb, 1), lambda i: (i, 0)),
            pl.BlockSpec((1, dp), lambda i: (0, 0)),
            pl.BlockSpec((1, dp), lambda i: (0, 0)),
        ],
        out_specs=pl.BlockSpec((rb, 1), lambda i: (i, 0)),
        out_shape=jax.ShapeDtypeStruct((n, 1), jnp.float32),
    )(a0, a1, g1, dinv, b1p, W2p)

    s0, s1 = _sc_agg(nb, npad, 1)(ep, g2.reshape(n), z1)

    out = pl.pallas_call(
        _tc3_body,
        grid=grid,
        in_specs=[
            pl.BlockSpec((rb, 1), lambda i: (i, 0)),
            pl.BlockSpec((rb, 1), lambda i: (i, 0)),
            pl.BlockSpec((rb, 1), lambda i: (i, 0)),
            pl.BlockSpec((rb, 1), lambda i: (i, 0)),
            pl.BlockSpec((1, 1), lambda i: (0, 0)),
        ],
        out_specs=pl.BlockSpec((rb, 1), lambda i: (i, 0)),
        out_shape=jax.ShapeDtypeStruct((n, 1), jnp.float32),
    )(s0[:n, None], s1[:n, None], g2, dinv, b2.reshape(1, 1))

    return out
